# baseline (speedup 1.0000x reference)
import functools

import jax
import jax.numpy as jnp
import numpy as np
from jax import lax
from jax.experimental import pallas as pl
from jax.experimental.pallas import tpu as pltpu
from jax.experimental.pallas import tpu_sc as plsc

N = 10000
E = 320000
D = 128
NP = 10240
NC = 2
NS = 16
NW = NC * NS
K = 128
RHI = 79
RLO = 78
NHI = 4
KA = 64
BCH = 16
BE = KA * BCH
NB = 9
NBUF = 5
NPA = 10240
RPW = NPA // NS

_mesh = plsc.VectorSubcoreMesh(core_axis_name="c", subcore_axis_name="s",
                               num_cores=NC, num_subcores=NS)


@functools.partial(
    pl.kernel,
    out_type=jax.ShapeDtypeStruct((NC * NP,), jnp.float32),
    mesh=_mesh,
    scratch_types=[
        pltpu.VMEM((RHI, K), jnp.int32),
        pltpu.VMEM((2, RHI * K), jnp.int32),
        pltpu.VMEM((K,), jnp.float32),
        pltpu.VMEM((NP // NS,), jnp.float32),
        pltpu.VMEM_SHARED((NP,), jnp.float32),
        pltpu.SemaphoreType.DMA,
    ],
)
def _deg_pass(ei_hbm, out_hbm, dst_v, stage, ones_v, zero_v, acc_sh, dsem):
    c = lax.axis_index("c")
    s = lax.axis_index("s")
    wid = c * NS + s
    hi = wid < NHI
    e0 = jnp.where(hi, wid * (RHI * K), NHI * RHI * K + (wid - NHI) * (RLO * K))
    nch = jnp.where(hi, RHI, RLO)
    pltpu.sync_copy(ei_hbm.at[pl.ds(0, 2), pl.ds(e0, RLO * K)],
                    stage.at[pl.ds(0, 2), pl.ds(0, RLO * K)])

    @pl.when(hi)
    def _extra():
        pltpu.sync_copy(ei_hbm.at[pl.ds(0, 2), pl.ds(e0 + RLO * K, K)],
                        stage.at[pl.ds(0, 2), pl.ds(RLO * K, K)])

    for i in range(K // 16):
        ones_v[pl.ds(16 * i, 16)] = jnp.ones((16,), jnp.float32)

    def rbody(r, carry):
        for q in range(K // 16):
            dst_v[r, pl.ds(16 * q, 16)] = stage[1, pl.ds(r * K + 16 * q, 16)]
        return carry

    lax.fori_loop(0, nch, rbody, 0)

    def zbody(i, carry):
        zero_v[pl.ds(i * 16, 16)] = jnp.zeros((16,), jnp.float32)
        return carry

    lax.fori_loop(0, NP // NS // 16, zbody, 0)
    pltpu.sync_copy(zero_v, acc_sh.at[pl.ds(s * (NP // NS), NP // NS)])
    plsc.subcore_barrier()

    def fire(j, carry):
        pltpu.async_copy(ones_v, acc_sh.at[dst_v.at[j]], dsem, add=True)
        return carry

    lax.fori_loop(0, nch, fire, 0)

    def drain(j, carry):
        pltpu.make_async_copy(ones_v, acc_sh.at[dst_v.at[j]], dsem).wait()
        return carry

    lax.fori_loop(0, nch, drain, 0)
    plsc.subcore_barrier()
    pltpu.sync_copy(acc_sh.at[pl.ds(s * (NP // NS), NP // NS)], zero_v)
    pltpu.sync_copy(zero_v,
                    out_hbm.at[pl.ds(c * NP + s * (NP // NS), NP // NS)])


@functools.partial(
    pl.kernel,
    out_type=jax.ShapeDtypeStruct((NC * NPA, D), jnp.float32),
    mesh=_mesh,
    scratch_types=[
        pltpu.VMEM((2, BE), jnp.int32),
        pltpu.VMEM((2, BE), jnp.int32),
        pltpu.VMEM((BCH, KA), jnp.int32),
        pltpu.VMEM((BCH, KA), jnp.int32),
        pltpu.VMEM((KA, D), jnp.float32),
        pltpu.VMEM((KA, D), jnp.float32),
        pltpu.VMEM((KA, D), jnp.float32),
        pltpu.VMEM((KA, D), jnp.float32),
        pltpu.VMEM((KA, D), jnp.float32),
        pltpu.VMEM_SHARED((NPA, D), jnp.float32),
        pltpu.SemaphoreType.DMA,
        pltpu.SemaphoreType.DMA,
        pltpu.SemaphoreType.DMA,
        pltpu.SemaphoreType.DMA,
        pltpu.SemaphoreType.DMA,
        pltpu.SemaphoreType.DMA,
        pltpu.SemaphoreType.DMA,
        pltpu.SemaphoreType.DMA,
        pltpu.SemaphoreType.DMA,
        pltpu.SemaphoreType.DMA,
        pltpu.SemaphoreType.DMA,
    ],
)
def _agg_pass(g_hbm, ei_hbm, out_hbm, stage_a, stage_b, srcb, dstb,
              rows0, rows1, rows2, rows3, rows4, acc_sh, gs0, gs1, gs2, gs3,
              gs4, ss0, ss1, ss2, ss3, ss4, isem):
    c = lax.axis_index("c")
    s = lax.axis_index("s")
    wid = c * NS + s
    rows = (rows0, rows1, rows2, rows3, rows4)
    gs = (gs0, gs1, gs2, gs3, gs4)
    ss = (ss0, ss1, ss2, ss3, ss4)

    def zbody(i, carry):
        for jj in range(D // 16):
            rows0[i, pl.ds(jj * 16, 16)] = jnp.zeros((16,), jnp.float32)
            rows1[i, pl.ds(jj * 16, 16)] = jnp.zeros((16,), jnp.float32)
        return carry

    lax.fori_loop(0, KA, zbody, 0)
    for t in range(RPW // KA // 2):
        pltpu.sync_copy(rows0, acc_sh.at[pl.ds(s * RPW + (2 * t) * KA, KA)])
        pltpu.sync_copy(rows1,
                        acc_sh.at[pl.ds(s * RPW + (2 * t + 1) * KA, KA)])
    plsc.subcore_barrier()

    def _g(b, buf):
        return pltpu.async_copy(g_hbm.at[srcb.at[b]], rows[buf], gs[buf])

    def _wg(b, buf):
        pltpu.make_async_copy(g_hbm.at[srcb.at[b]], rows[buf],
                              gs[buf]).wait()

    def _s(b, buf):
        return pltpu.async_copy(rows[buf], acc_sh.at[dstb.at[b]], ss[buf],
                                add=True)

    def _ws(b, buf):
        pltpu.make_async_copy(rows[buf], acc_sh.at[dstb.at[b]],
                              ss[buf]).wait()

    def _repack1(ch, stg):
        for q in range(KA // 16):
            o = KA * ch + 16 * q
            srcb[ch, pl.ds(16 * q, 16)] = stg[0, pl.ds(o, 16)]
            dstb[ch, pl.ds(16 * q, 16)] = stg[1, pl.ds(o, 16)]

    def _pipeline(nch, stg):
        for p in range(NBUF - 1):
            _repack1(p, stg)
            _g(p, p)
        for b in range(nch):
            _wg(b, b % NBUF)
            _s(b, b % NBUF)
            nb = b + NBUF - 1
            if nb < nch:
                if b >= 1:
                    _ws(b - 1, (b - 1) % NBUF)
                _repack1(nb, stg)
                _g(nb, nb % NBUF)
        for b in range(max(0, nch - NBUF), nch):
            _ws(b, b % NBUF)

    hi = wid < NHI
    e0 = jnp.where(hi, wid * (RHI * K), NHI * RHI * K + (wid - NHI) * (RLO * K))

    def _fire_idx(base, stg):
        pltpu.async_copy(ei_hbm.at[pl.ds(0, 2), pl.ds(base, BE)], stg,
                         isem)

    def _wait_idx(stg):
        pltpu.make_async_copy(ei_hbm.at[pl.ds(0, 2), pl.ds(0, BE)], stg,
                              isem).wait()

    _fire_idx(e0, stage_a)

    def block2(i, carry):
        b0 = e0 + (2 * i) * BE
        _wait_idx(stage_a)
        _fire_idx(b0 + BE, stage_b)
        _pipeline(BCH, stage_a)
        _wait_idx(stage_b)
        _fire_idx(b0 + 2 * BE, stage_a)
        _pipeline(BCH, stage_b)
        return carry

    lax.fori_loop(0, (NB - 1) // 2, block2, 0)

    _wait_idx(stage_a)
    tbase = e0 + NB * BE
    pltpu.async_copy(ei_hbm.at[pl.ds(0, 2), pl.ds(tbase, (RLO - 72) * K)],
                     stage_b.at[pl.ds(0, 2), pl.ds(0, (RLO - 72) * K)], isem)
    _pipeline(BCH, stage_a)
    pltpu.make_async_copy(ei_hbm.at[pl.ds(0, 2), pl.ds(0, (RLO - 72) * K)],
                          stage_b.at[pl.ds(0, 2), pl.ds(0, (RLO - 72) * K)],
                          isem).wait()

    @pl.when(hi)
    def _tail_hi():
        pltpu.sync_copy(ei_hbm.at[pl.ds(0, 2), pl.ds(tbase + (RLO - 72) * K, K)],
                        stage_b.at[pl.ds(0, 2), pl.ds((RLO - 72) * K, K)])
        _pipeline((RHI - 72) * 2, stage_b)

    @pl.when(jnp.logical_not(hi))
    def _tail_lo():
        _pipeline((RLO - 72) * 2, stage_b)
    plsc.subcore_barrier()
    pltpu.sync_copy(acc_sh.at[pl.ds(s * RPW, RPW)],
                    out_hbm.at[pl.ds(c * NPA + s * RPW, RPW)])


def _dense0_body(x_ref, w_ref, h_ref):
    h_ref[...] = lax.dot_general(x_ref[...], w_ref[...],
                                 (((1,), (1,)), ((), ())),
                                 precision=lax.Precision.HIGHEST,
                                 preferred_element_type=jnp.float32)


_dense0 = pl.pallas_call(
    _dense0_body,
    out_shape=jax.ShapeDtypeStruct((N, D), jnp.float32),
)


def _dis_col(d_ref):
    deg = d_ref[0:1, 0:N] + d_ref[1:2, 0:N] + 1.0
    return lax.transpose(lax.rsqrt(deg), (1, 0))


def _dense1_body(h_ref, d_ref, g_ref):
    g_ref[...] = _dis_col(d_ref) * h_ref[...]


_dense1 = pl.pallas_call(
    _dense1_body,
    out_shape=jax.ShapeDtypeStruct((N, D), jnp.float32),
)


def _dense2_body(acc_ref, g_ref, d_ref, b_ref, o_ref):
    tot = acc_ref[0:N, :] + acc_ref[NPA:NPA + N, :] + g_ref[...]
    o_ref[...] = jnp.maximum(_dis_col(d_ref) * tot + b_ref[...], 0.0)


_dense2 = pl.pallas_call(
    _dense2_body,
    out_shape=jax.ShapeDtypeStruct((N, D), jnp.float32),
)


def kernel(x, edge_index, W, b):
    ei = edge_index.astype(jnp.int32)

    h = _dense0(x, W)
    degf = _deg_pass(ei)
    d2 = degf.reshape(NC, NP)

    g = _dense1(h, d2)
    accf = _agg_pass(g, ei)
    out = _dense2(accf, g, d2, b.reshape(1, D))
    return out
